# baseline (speedup 1.0000x reference)
import jax
import jax.numpy as jnp
from jax.experimental import pallas as pl
from jax.experimental.pallas import tpu as pltpu

_NSEM = 16

_ANY = pl.ANY


def _row_perm_kernel(si_ref, x_ref, o_ref, sem):
    C, V = si_ref.shape

    def _copy(i):
        c = i // V
        w = i % V
        u = si_ref[c, w]
        return pltpu.make_async_copy(
            x_ref.at[:, pl.ds(u, 1), pl.ds(c, 1), :],
            o_ref.at[:, pl.ds(w, 1), pl.ds(c, 1), :],
            sem.at[i % _NSEM],
        )

    def _issue(i, carry):
        _copy(i).start()
        return carry

    def _drain(i, carry):
        _copy(i).wait()
        return carry

    n = C * V
    jax.lax.fori_loop(0, n, _issue, 0)
    jax.lax.fori_loop(0, n, _drain, 0)


def kernel(x, shift_indices):
    B, C, T, V = x.shape
    xt = jnp.transpose(x, (0, 3, 1, 2))
    si = shift_indices.astype(jnp.int32)

    grid_spec = pltpu.PrefetchScalarGridSpec(
        num_scalar_prefetch=1,
        grid=(1,),
        in_specs=[pl.BlockSpec(memory_space=_ANY)],
        out_specs=pl.BlockSpec(memory_space=_ANY),
        scratch_shapes=[pltpu.SemaphoreType.DMA((_NSEM,))],
    )
    out_t = pl.pallas_call(
        _row_perm_kernel,
        grid_spec=grid_spec,
        out_shape=jax.ShapeDtypeStruct((B, V, C, T), x.dtype),
    )(si, xt)
    return jnp.transpose(out_t, (0, 2, 3, 1))


# speedup vs baseline: 8.5181x; 8.5181x over previous
import jax
import jax.numpy as jnp
from jax.experimental import pallas as pl

_CBLK = 32


def _sel_kernel(si_ref, x_ref, o_ref):
    xv = x_ref[0]
    si = si_ref[...]
    V = xv.shape[0]
    for w in range(V):
        col = si[:, w:w + 1]
        acc = xv[0]
        for u in range(1, V):
            acc = jnp.where(col == u, xv[u], acc)
        o_ref[0, w] = acc


def kernel(x, shift_indices):
    B, C, T, V = x.shape
    xt = jnp.transpose(x, (0, 3, 1, 2))
    si = shift_indices.astype(jnp.int32)
    grid = (C // _CBLK, B)
    out_t = pl.pallas_call(
        _sel_kernel,
        grid=grid,
        in_specs=[
            pl.BlockSpec((_CBLK, V), lambda j, b: (j, 0)),
            pl.BlockSpec((1, V, _CBLK, T), lambda j, b: (b, 0, j, 0)),
        ],
        out_specs=pl.BlockSpec((1, V, _CBLK, T), lambda j, b: (b, 0, j, 0)),
        out_shape=jax.ShapeDtypeStruct((B, V, C, T), x.dtype),
    )(si, xt)
    return jnp.transpose(out_t, (0, 2, 3, 1))


# speedup vs baseline: 33.9915x; 3.9905x over previous
import functools
import jax
import jax.numpy as jnp
from jax import lax
from jax.experimental import pallas as pl
from jax.experimental.pallas import tpu as pltpu
from jax.experimental.pallas import tpu_sc as plsc

_R = 320


def _make_sc_kernel(N, D):
    info = plsc.get_sparse_core_info()
    NC, NS = info.num_cores, info.num_subcores
    NW = NC * NS
    per_w = N // NW
    n_chunks = per_w // _R
    mesh = plsc.VectorSubcoreMesh(core_axis_name="c", subcore_axis_name="s")

    @functools.partial(
        pl.kernel, mesh=mesh,
        out_type=jax.ShapeDtypeStruct((N, D), jnp.float32),
        scratch_types=[
            pltpu.VMEM((_R,), jnp.int32),
            pltpu.VMEM((_R, D), jnp.float32),
            pltpu.SemaphoreType.DMA,
        ],
    )
    def k(table_hbm, idx_hbm, out_hbm, idx_v, rows_v, sem):
        wid = lax.axis_index("s") * NC + lax.axis_index("c")
        base = wid * per_w
        for j in range(n_chunks):
            off = base + j * _R
            pltpu.sync_copy(idx_hbm.at[pl.ds(off, _R)], idx_v)
            pltpu.async_copy(table_hbm.at[idx_v], rows_v, sem).wait()
            pltpu.sync_copy(rows_v, out_hbm.at[pl.ds(off, _R)])

    return k


def kernel(x, shift_indices):
    B, C, T, V = x.shape
    N = B * V * C
    xt = jnp.transpose(x, (0, 3, 1, 2)).reshape(N, T)
    si = shift_indices.astype(jnp.int32)
    ridx = (si.T[None, :, :] * C
            + jnp.arange(C, dtype=jnp.int32)[None, None, :]
            + (jnp.arange(B, dtype=jnp.int32) * (V * C))[:, None, None]
            ).reshape(N)
    out2 = _make_sc_kernel(N, T)(xt, ridx)
    return jnp.transpose(out2.reshape(B, V, C, T), (0, 2, 3, 1))


# speedup vs baseline: 35.0024x; 1.0297x over previous
import functools
import jax
import jax.numpy as jnp
from jax import lax
from jax.experimental import pallas as pl
from jax.experimental.pallas import tpu as pltpu
from jax.experimental.pallas import tpu_sc as plsc

_R = 400


def _make_sc_kernel(N, D):
    info = plsc.get_sparse_core_info()
    NC, NS = info.num_cores, info.num_subcores
    NW = NC * NS
    per_w = N // NW
    n_chunks = per_w // _R
    mesh = plsc.VectorSubcoreMesh(core_axis_name="c", subcore_axis_name="s")

    @functools.partial(
        pl.kernel, mesh=mesh,
        out_type=jax.ShapeDtypeStruct((N, D), jnp.float32),
        scratch_types=[
            pltpu.VMEM((per_w,), jnp.int32),
            pltpu.VMEM((_R, D), jnp.float32),
            pltpu.SemaphoreType.DMA,
        ],
    )
    def k(table_hbm, idx_hbm, out_hbm, idx_v, rows_v, sem):
        wid = lax.axis_index("s") * NC + lax.axis_index("c")
        base = wid * per_w
        pltpu.sync_copy(idx_hbm.at[pl.ds(base, per_w)], idx_v)
        for j in range(n_chunks):
            pltpu.async_copy(
                table_hbm.at[idx_v.at[pl.ds(j * _R, _R)]], rows_v, sem
            ).wait()
            pltpu.sync_copy(rows_v, out_hbm.at[pl.ds(base + j * _R, _R)])

    return k


def kernel(x, shift_indices):
    B, C, T, V = x.shape
    N = B * V * C
    xt = jnp.transpose(x, (0, 3, 1, 2)).reshape(N, T)
    si = shift_indices.astype(jnp.int32)
    ridx = (si.T[None, :, :] * C
            + jnp.arange(C, dtype=jnp.int32)[None, None, :]
            + (jnp.arange(B, dtype=jnp.int32) * (V * C))[:, None, None]
            ).reshape(N)
    out2 = _make_sc_kernel(N, T)(xt, ridx)
    return jnp.transpose(out2.reshape(B, V, C, T), (0, 2, 3, 1))


# speedup vs baseline: 35.5652x; 1.0161x over previous
import functools
import jax
import jax.numpy as jnp
from jax import lax
from jax.experimental import pallas as pl
from jax.experimental.pallas import tpu as pltpu
from jax.experimental.pallas import tpu_sc as plsc

_R = 160


def _make_sc_kernel(N, D):
    info = plsc.get_sparse_core_info()
    NC, NS = info.num_cores, info.num_subcores
    NW = NC * NS
    per_w = N // NW
    n_chunks = per_w // _R
    mesh = plsc.VectorSubcoreMesh(core_axis_name="c", subcore_axis_name="s")

    @functools.partial(
        pl.kernel, mesh=mesh,
        out_type=jax.ShapeDtypeStruct((N, D), jnp.float32),
        scratch_types=[
            pltpu.VMEM((per_w,), jnp.int32),
            pltpu.VMEM((_R, D), jnp.float32),
            pltpu.VMEM((_R, D), jnp.float32),
            pltpu.SemaphoreType.DMA,
            pltpu.SemaphoreType.DMA,
            pltpu.SemaphoreType.DMA,
        ],
    )
    def k(table_hbm, idx_hbm, out_hbm, idx_v, rows_a, rows_b, gsem, ssem_a,
          ssem_b):
        wid = lax.axis_index("s") * NC + lax.axis_index("c")
        base = wid * per_w
        pltpu.sync_copy(idx_hbm.at[pl.ds(base, per_w)], idx_v)
        bufs = (rows_a, rows_b)
        ssems = (ssem_a, ssem_b)

        def _scatter(i):
            return pltpu.make_async_copy(
                bufs[i % 2], out_hbm.at[pl.ds(base + i * _R, _R)], ssems[i % 2]
            )

        for i in range(n_chunks):
            if i >= 2:
                _scatter(i - 2).wait()
            pltpu.async_copy(
                table_hbm.at[idx_v.at[pl.ds(i * _R, _R)]], bufs[i % 2], gsem
            ).wait()
            _scatter(i).start()
        _scatter(n_chunks - 2).wait()
        _scatter(n_chunks - 1).wait()

    return k


def kernel(x, shift_indices):
    B, C, T, V = x.shape
    N = B * V * C
    xt = jnp.transpose(x, (0, 3, 1, 2)).reshape(N, T)
    si = shift_indices.astype(jnp.int32)
    ridx = (si.T[None, :, :] * C
            + jnp.arange(C, dtype=jnp.int32)[None, None, :]
            + (jnp.arange(B, dtype=jnp.int32) * (V * C))[:, None, None]
            ).reshape(N)
    out2 = _make_sc_kernel(N, T)(xt, ridx)
    return jnp.transpose(out2.reshape(B, V, C, T), (0, 2, 3, 1))
